# Initial kernel scaffold; baseline (speedup 1.0000x reference)
import jax
import jax.numpy as jnp
from jax.experimental import pallas as pl


def kernel(x, e, edge_index, W_node, b_node, W_to, b_to, W_edge, b_edge):
    raise NotImplementedError("write your pallas kernel here")


# speedup vs baseline: 4.7745x; 4.7745x over previous
import functools

import jax
import jax.numpy as jnp
from jax import lax
from jax.experimental import pallas as pl
from jax.experimental.pallas import tpu as pltpu
from jax.experimental.pallas import tpu_sc as plsc

_NC = 2
_NS = 16
_NW = _NC * _NS
_CH = 128

_R = 16


def _node_body(x_ref, wn_ref, bn_ref, wt_ref, bt_ref, ux_ref, vx_ref):
    xb = x_ref[...]
    ux_ref[...] = (
        jnp.dot(xb, wn_ref[...], preferred_element_type=jnp.float32) + bn_ref[...]
    )
    vx_ref[...] = (
        jnp.dot(xb, wt_ref[...], preferred_element_type=jnp.float32) + bt_ref[...]
    )


def _node_embeddings(x2, wn_t, bn, wt_t, bt):
    bn_rows, h = x2.shape
    blk = 2000 if bn_rows % 2000 == 0 else bn_rows
    grid = bn_rows // blk
    return pl.pallas_call(
        _node_body,
        grid=(grid,),
        in_specs=[
            pl.BlockSpec((blk, h), lambda i: (i, 0)),
            pl.BlockSpec((h, h), lambda i: (0, 0)),
            pl.BlockSpec((1, h), lambda i: (0, 0)),
            pl.BlockSpec((h, h), lambda i: (0, 0)),
            pl.BlockSpec((1, h), lambda i: (0, 0)),
        ],
        out_specs=[
            pl.BlockSpec((blk, h), lambda i: (i, 0)),
            pl.BlockSpec((blk, h), lambda i: (i, 0)),
        ],
        out_shape=[
            jax.ShapeDtypeStruct((bn_rows, h), jnp.float32),
            jax.ShapeDtypeStruct((bn_rows, h), jnp.float32),
        ],
    )(x2, wn_t, bn, wt_t, bt)


def _sc_gather(table, idx2):
    nchunks = idx2.shape[0]
    h = table.shape[1]
    total = nchunks * _CH
    mesh = plsc.VectorSubcoreMesh(
        core_axis_name="c", subcore_axis_name="s", num_cores=_NC, num_subcores=_NS
    )

    @functools.partial(
        pl.kernel,
        out_type=jax.ShapeDtypeStruct((total, h), jnp.float32),
        mesh=mesh,
        scratch_types=[
            pltpu.VMEM((_CH,), jnp.int32),
            pltpu.VMEM((_CH, h), jnp.float32),
            pltpu.SemaphoreType.DMA,
        ],
    )
    def gather_k(table_hbm, idx_hbm, out_hbm, idx_v, rows_v, sem):
        wid = lax.axis_index("s") * _NC + lax.axis_index("c")

        def body(i, carry):
            c = wid + i * _NW
            pltpu.sync_copy(idx_hbm.at[c], idx_v)
            pltpu.async_copy(table_hbm.at[idx_v], rows_v, sem).wait()
            pltpu.sync_copy(rows_v, out_hbm.at[pl.ds(c * _CH, _CH)])
            return carry

        n_i = (nchunks - wid + _NW - 1) // _NW
        lax.fori_loop(0, n_i, body, 0)

    return gather_k(table, idx2)


def _edge_body(e_ref, vxg_ref, ux_ref, we_ref, be_ref, st_ref, out_ref):
    ve = (
        jnp.dot(e_ref[...], we_ref[...], preferred_element_type=jnp.float32)
        + be_ref[...]
    )
    ex = jnp.exp(ve)
    st = st_ref[...]
    denom = jnp.dot(st, ex, preferred_element_type=jnp.float32)
    num = jnp.dot(st, ex * vxg_ref[...], preferred_element_type=jnp.float32)
    out_ref[...] = ux_ref[...] + num / denom


def _edge_pass(e2, vxg, ux, we_t, be, st, k):
    rows, h = e2.shape
    rb = _R * k
    grid = rows // rb
    return pl.pallas_call(
        _edge_body,
        grid=(grid,),
        in_specs=[
            pl.BlockSpec((rb, h), lambda i: (i, 0)),
            pl.BlockSpec((rb, h), lambda i: (i, 0)),
            pl.BlockSpec((_R, h), lambda i: (i, 0)),
            pl.BlockSpec((h, h), lambda i: (0, 0)),
            pl.BlockSpec((1, h), lambda i: (0, 0)),
            pl.BlockSpec((_R, rb), lambda i: (0, 0)),
        ],
        out_specs=pl.BlockSpec((_R, h), lambda i: (i, 0)),
        out_shape=jax.ShapeDtypeStruct((rows // k, h), jnp.float32),
    )(e2, vxg, ux, we_t, be, st)


def kernel(x, e, edge_index, W_node, b_node, W_to, b_to, W_edge, b_edge):
    b, n, h = x.shape
    nk = e.shape[1]
    k = nk // n

    x2 = x.reshape(b * n, h)
    e2 = e.reshape(b * nk, h)

    gidx = (
        edge_index.astype(jnp.int32) + (jnp.arange(b, dtype=jnp.int32) * n)[:, None]
    ).reshape(-1)
    idx2 = gidx.reshape((b * nk) // _CH, _CH)

    ux, vx = _node_embeddings(x2, W_node.T, b_node[None], W_to.T, b_to[None])
    vxg = _sc_gather(vx, idx2)

    st = (jnp.arange(_R)[:, None] == (jnp.arange(_R * k) // k)[None, :]).astype(
        jnp.float32
    )

    out2 = _edge_pass(e2, vxg, ux, W_edge.T, b_edge[None], st, k)
    return out2.reshape(b, n, h)


# speedup vs baseline: 5.2011x; 1.0893x over previous
import functools

import jax
import jax.numpy as jnp
from jax import lax
from jax.experimental import pallas as pl
from jax.experimental.pallas import tpu as pltpu
from jax.experimental.pallas import tpu_sc as plsc

_NC = 2
_NS = 16
_NW = _NC * _NS
_CH = 128
_NBUF = 4

_R = 32


def _node_body(x_ref, wn_ref, bn_ref, wt_ref, bt_ref, ux_ref, vx_ref):
    xb = x_ref[...]
    ux_ref[...] = (
        jnp.dot(xb, wn_ref[...], preferred_element_type=jnp.float32) + bn_ref[...]
    )
    vx_ref[...] = (
        jnp.dot(xb, wt_ref[...], preferred_element_type=jnp.float32) + bt_ref[...]
    )


def _node_embeddings(x2, wn_t, bn, wt_t, bt):
    bn_rows, h = x2.shape
    blk = 2000 if bn_rows % 2000 == 0 else bn_rows
    grid = bn_rows // blk
    return pl.pallas_call(
        _node_body,
        grid=(grid,),
        in_specs=[
            pl.BlockSpec((blk, h), lambda i: (i, 0)),
            pl.BlockSpec((h, h), lambda i: (0, 0)),
            pl.BlockSpec((1, h), lambda i: (0, 0)),
            pl.BlockSpec((h, h), lambda i: (0, 0)),
            pl.BlockSpec((1, h), lambda i: (0, 0)),
        ],
        out_specs=[
            pl.BlockSpec((blk, h), lambda i: (i, 0)),
            pl.BlockSpec((blk, h), lambda i: (i, 0)),
        ],
        out_shape=[
            jax.ShapeDtypeStruct((bn_rows, h), jnp.float32),
            jax.ShapeDtypeStruct((bn_rows, h), jnp.float32),
        ],
    )(x2, wn_t, bn, wt_t, bt)


def _sc_gather(table, idxp):
    nchunks = idxp.shape[0]
    h = table.shape[1]
    total = nchunks * _CH
    per_tile = nchunks // _NW
    n_outer = per_tile // _NBUF
    mesh = plsc.VectorSubcoreMesh(
        core_axis_name="c", subcore_axis_name="s", num_cores=_NC, num_subcores=_NS
    )

    @functools.partial(
        pl.kernel,
        out_type=jax.ShapeDtypeStruct((total, h), jnp.float32),
        mesh=mesh,
        scratch_types=[
            pltpu.VMEM((_NBUF, _CH), jnp.int32),
            pltpu.VMEM((_NBUF, _CH, h), jnp.float32),
            pltpu.SemaphoreType.DMA((_NBUF,)),
            pltpu.SemaphoreType.DMA((_NBUF,)),
        ],
    )
    def gather_k(table_hbm, idx_hbm, out_hbm, idx_v, rows_v, isem, gsem):
        wid = lax.axis_index("s") * _NC + lax.axis_index("c")

        def body(i, carry):
            base = wid + i * _NBUF * _NW
            for u in range(_NBUF):
                pltpu.async_copy(idx_hbm.at[base + u * _NW], idx_v.at[u], isem.at[u])
            for u in range(_NBUF):
                pltpu.make_async_copy(
                    idx_hbm.at[base + u * _NW], idx_v.at[u], isem.at[u]
                ).wait()
                pltpu.async_copy(table_hbm.at[idx_v.at[u]], rows_v.at[u], gsem.at[u])
            for u in range(_NBUF):
                c = base + u * _NW
                pltpu.make_async_copy(
                    table_hbm.at[idx_v.at[u]], rows_v.at[u], gsem.at[u]
                ).wait()
                pltpu.sync_copy(rows_v.at[u], out_hbm.at[pl.ds(c * _CH, _CH)])
            return carry

        lax.fori_loop(0, n_outer, body, 0)

    return gather_k(table, idxp)


def _edge_body(e_ref, vxg_ref, ux_ref, we_ref, be_ref, st_ref, out_ref):
    ve = (
        jnp.dot(e_ref[...], we_ref[...], preferred_element_type=jnp.float32)
        + be_ref[...]
    )
    ex = jnp.exp(ve)
    both = jnp.concatenate(
        [ex.astype(jnp.bfloat16), (ex * vxg_ref[...]).astype(jnp.bfloat16)], axis=1
    )
    sums = jnp.dot(st_ref[...], both, preferred_element_type=jnp.float32)
    h = out_ref.shape[1]
    out_ref[...] = ux_ref[...] + sums[:, h:] / sums[:, :h]


def _edge_pass(e2, vxg, ux, we_t, be, st, k):
    rows, h = e2.shape
    rb = _R * k
    grid = rows // rb
    return pl.pallas_call(
        _edge_body,
        grid=(grid,),
        in_specs=[
            pl.BlockSpec((rb, h), lambda i: (i, 0)),
            pl.BlockSpec((rb, h), lambda i: (i, 0)),
            pl.BlockSpec((_R, h), lambda i: (i, 0)),
            pl.BlockSpec((h, h), lambda i: (0, 0)),
            pl.BlockSpec((1, h), lambda i: (0, 0)),
            pl.BlockSpec((_R, rb), lambda i: (0, 0)),
        ],
        out_specs=pl.BlockSpec((_R, h), lambda i: (i, 0)),
        out_shape=jax.ShapeDtypeStruct((rows // k, h), jnp.float32),
    )(e2, vxg, ux, we_t, be, st)


def kernel(x, e, edge_index, W_node, b_node, W_to, b_to, W_edge, b_edge):
    b, n, h = x.shape
    nk = e.shape[1]
    k = nk // n

    x2 = x.reshape(b * n, h)
    e2 = e.reshape(b * nk, h)

    gidx = (
        edge_index.astype(jnp.int32) + (jnp.arange(b, dtype=jnp.int32) * n)[:, None]
    ).reshape(-1)
    total_pad = -(b * nk) % (_CH * _NBUF * _NW)
    idxp = jnp.pad(gidx, (0, total_pad)).reshape(-1, _CH)

    ux, vx = _node_embeddings(x2, W_node.T, b_node[None], W_to.T, b_to[None])
    vxg = _sc_gather(vx, idxp)

    st = (jnp.arange(_R)[:, None] == (jnp.arange(_R * k) // k)[None, :]).astype(
        jnp.bfloat16
    )

    out2 = _edge_pass(e2, vxg, ux, W_edge.T, b_edge[None], st, k)
    return out2.reshape(b, n, h)
